# baseline (speedup 1.0000x reference)
import functools

import jax
import jax.numpy as jnp
from jax import lax
from jax.experimental import pallas as pl
from jax.experimental.pallas import tpu as pltpu
from jax.experimental.pallas import tpu_sc as plsc

_B = 16384
_D = 16
_H = 32
_NC, _NS = 2, 16
_NW = _NC * _NS
_BPW = _B // _NW
_CH = 256
_NCHUNK = _BPW // _CH
_ROWS128 = 1000000 * _D // 128

_sc_mesh = plsc.VectorSubcoreMesh(
    core_axis_name="c", subcore_axis_name="s", num_cores=_NC, num_subcores=_NS
)


def _extract_chunk(ids_v, off, buf, out_v):
    def group(g, _):
        ids = ids_v[pl.ds(off + g * 16, 16)]
        col0 = (ids & 7) * 16
        row_local = g * 16 + lax.iota(jnp.int32, 16)
        out0 = (off + row_local) * 16
        for k in range(_D):
            v = plsc.load_gather(buf, [row_local, col0 + k])
            plsc.store_scatter(out_v, [out0 + k], v)
        return 0
    lax.fori_loop(0, _CH // 16, group, 0)


@functools.partial(
    pl.kernel,
    out_type=(
        jax.ShapeDtypeStruct((_B * _D,), jnp.float32),
        jax.ShapeDtypeStruct((_B * _D,), jnp.float32),
    ),
    mesh=_sc_mesh,
    compiler_params=pltpu.CompilerParams(needs_layout_passes=False),
    scratch_types=[
        pltpu.VMEM((_BPW,), jnp.int32),
        pltpu.VMEM((_BPW,), jnp.int32),
        pltpu.VMEM((_BPW,), jnp.int32),
        pltpu.VMEM((_BPW,), jnp.int32),
        pltpu.VMEM((2, _CH, 128), jnp.float32),
        pltpu.VMEM((_BPW * _D,), jnp.float32),
        pltpu.VMEM((_BPW * _D,), jnp.float32),
        pltpu.SemaphoreType.DMA,
        pltpu.SemaphoreType.DMA,
    ],
)
def _sc_gather(uid_hbm, iid_hbm, utab_hbm, itab_hbm, u_out, i_out,
               uids_v, iids_v, uq_v, iq_v, rows_v, uout_v, iout_v,
               sem0, sem1):
    wid = lax.axis_index("s") * _NC + lax.axis_index("c")
    base = wid * _BPW
    pltpu.sync_copy(uid_hbm.at[pl.ds(base, _BPW)], uids_v)
    pltpu.sync_copy(iid_hbm.at[pl.ds(base, _BPW)], iids_v)

    def qbody(g, _):
        sl = pl.ds(g * 16, 16)
        uq_v[sl] = lax.shift_right_logical(uids_v[sl], 3)
        iq_v[sl] = lax.shift_right_logical(iids_v[sl], 3)
        return 0
    lax.fori_loop(0, _BPW // 16, qbody, 0)

    chunks = [(utab_hbm, uq_v, uids_v, uout_v, c * _CH) for c in range(_NCHUNK)]
    chunks += [(itab_hbm, iq_v, iids_v, iout_v, c * _CH) for c in range(_NCHUNK)]
    sems = [sem0, sem1]

    def fire(c, slot):
        tab, q_v, _, _, off = chunks[c]
        return pltpu.async_copy(
            tab.at[q_v.at[pl.ds(off, _CH)]], rows_v.at[slot], sems[slot])

    copies = {0: fire(0, 0)}
    if len(chunks) > 1:
        copies[1] = fire(1, 1)
    for c in range(len(chunks)):
        copies[c].wait()
        _, _, ids_v, out_v, off = chunks[c]
        _extract_chunk(ids_v, off, rows_v.at[c % 2], out_v)
        if c + 2 < len(chunks):
            copies[c + 2] = fire(c + 2, c % 2)

    pltpu.sync_copy(uout_v, u_out.at[pl.ds(base * _D, _BPW * _D)])
    pltpu.sync_copy(iout_v, i_out.at[pl.ds(base * _D, _BPW * _D)])


_BLK = 2048


def _mlp_body(u_ref, i_ref, w1u_ref, w1i_ref, b1_ref, w2_ref, b2_ref, out_ref):
    h = lax.dot_general(u_ref[...], w1u_ref[...], (((1,), (0,)), ((), ())),
                        preferred_element_type=jnp.float32)
    h += lax.dot_general(i_ref[...], w1i_ref[...], (((1,), (0,)), ((), ())),
                         preferred_element_type=jnp.float32)
    h = jnp.maximum(h + b1_ref[...], 0.0)
    out_ref[...] = jnp.sum(h * w2_ref[...], axis=1, keepdims=True) + b2_ref[...]


def _mlp(u, i, w1uT, w1iT, b1, W2, b2):
    return pl.pallas_call(
        _mlp_body,
        grid=(_B // _BLK,),
        in_specs=[
            pl.BlockSpec((_BLK, _D), lambda b: (b, 0)),
            pl.BlockSpec((_BLK, _D), lambda b: (b, 0)),
            pl.BlockSpec((_D, _H), lambda b: (0, 0)),
            pl.BlockSpec((_D, _H), lambda b: (0, 0)),
            pl.BlockSpec((1, _H), lambda b: (0, 0)),
            pl.BlockSpec((1, _H), lambda b: (0, 0)),
            pl.BlockSpec((1, 1), lambda b: (0, 0)),
        ],
        out_specs=pl.BlockSpec((_BLK, 1), lambda b: (b, 0)),
        out_shape=jax.ShapeDtypeStruct((_B, 1), jnp.float32),
    )(u, i, w1uT, w1iT, b1, W2, b2)


def kernel(user_ids, item_ids, user_table, item_table, W1, b1, W2, b2):
    user_ids = user_ids.astype(jnp.int32)
    item_ids = item_ids.astype(jnp.int32)
    utab128 = user_table.reshape(_ROWS128, 128)
    itab128 = item_table.reshape(_ROWS128, 128)
    u_flat, i_flat = _sc_gather(user_ids, item_ids, utab128, itab128)
    u = u_flat.reshape(_B, _D)
    i = i_flat.reshape(_B, _D)
    w1uT = W1[:, :_D].T
    w1iT = W1[:, _D:].T
    return _mlp(u, i, w1uT, w1iT, b1.reshape(1, _H), W2, b2.reshape(1, 1))


# speedup vs baseline: 5.1655x; 5.1655x over previous
import functools

import jax
import jax.numpy as jnp
from jax import lax
from jax.experimental import pallas as pl
from jax.experimental.pallas import tpu as pltpu
from jax.experimental.pallas import tpu_sc as plsc

_B = 16384
_D = 16
_H = 32
_NROWS = 1000000
_NC, _NS = 2, 16
_NW = _NC * _NS
_BPW = _B // _NW
_G = 16
_NG = _BPW // _G
_CMAX = _NROWS - 128

_sc_mesh = plsc.VectorSubcoreMesh(
    core_axis_name="c", subcore_axis_name="s", num_cores=_NC, num_subcores=_NS
)


@functools.partial(
    pl.kernel,
    out_type=(
        jax.ShapeDtypeStruct((_B * _D,), jnp.float32),
        jax.ShapeDtypeStruct((_B * _D,), jnp.float32),
    ),
    mesh=_sc_mesh,
    compiler_params=pltpu.CompilerParams(needs_layout_passes=False),
    scratch_types=[
        pltpu.VMEM((_BPW,), jnp.int32),
        pltpu.VMEM((_BPW,), jnp.int32),
        pltpu.VMEM((2, _G, _D, 128), jnp.float32),
        pltpu.VMEM((_BPW * _D,), jnp.float32),
        pltpu.VMEM((_BPW * _D,), jnp.float32),
        pltpu.SemaphoreType.DMA,
        pltpu.SemaphoreType.DMA,
    ],
)
def _sc_gather(uid_hbm, iid_hbm, utabT, itabT, u_out, i_out,
               uids_v, iids_v, slab_v, uout_v, iout_v, sem0, sem1):
    wid = lax.axis_index("s") * _NC + lax.axis_index("c")
    base = wid * _BPW
    pltpu.sync_copy(uid_hbm.at[pl.ds(base, _BPW)], uids_v)
    pltpu.sync_copy(iid_hbm.at[pl.ds(base, _BPW)], iids_v)
    sems = (sem0, sem1)
    iota = lax.iota(jnp.int32, _G)

    def fire(g, slot, tab, ids_v):
        ids = ids_v[pl.ds(g * _G, _G)]
        cvec = (ids >> 7) * 128
        for j in range(_G):
            c = pl.multiple_of(cvec[j], 128)
            pltpu.async_copy(tab.at[:, pl.ds(c, 128)],
                             slab_v.at[slot, j], sems[slot])

    def drain(slot, tab):
        for j in range(_G):
            pltpu.make_async_copy(tab.at[:, pl.ds(0, 128)],
                                  slab_v.at[slot, j], sems[slot]).wait()

    def extract(g, slot, ids_v, out_v):
        ids = ids_v[pl.ds(g * _G, _G)]
        col = ids & 127
        out0 = (g * _G + iota) * _D
        slab = slab_v.at[slot]
        for k in range(_D):
            v = plsc.load_gather(slab, [iota, jnp.full((_G,), k, jnp.int32),
                                        col])
            plsc.store_scatter(out_v, [out0 + k], v)

    def run_table(tab, ids_v, out_v):
        fire(0, 0, tab, ids_v)

        def body(p, _):
            g0 = 2 * p
            fire(g0 + 1, 1, tab, ids_v)
            drain(0, tab)
            extract(g0, 0, ids_v, out_v)

            @pl.when(p < _NG // 2 - 1)
            def _():
                fire(g0 + 2, 0, tab, ids_v)

            drain(1, tab)
            extract(g0 + 1, 1, ids_v, out_v)
            return 0

        lax.fori_loop(0, _NG // 2, body, 0)

    run_table(utabT, uids_v, uout_v)
    run_table(itabT, iids_v, iout_v)
    pltpu.sync_copy(uout_v, u_out.at[pl.ds(base * _D, _BPW * _D)])
    pltpu.sync_copy(iout_v, i_out.at[pl.ds(base * _D, _BPW * _D)])


_BLK = 2048


def _mlp_body(u_ref, i_ref, w1u_ref, w1i_ref, b1_ref, w2_ref, b2_ref, out_ref):
    h = lax.dot_general(u_ref[...], w1u_ref[...], (((1,), (0,)), ((), ())),
                        preferred_element_type=jnp.float32)
    h += lax.dot_general(i_ref[...], w1i_ref[...], (((1,), (0,)), ((), ())),
                         preferred_element_type=jnp.float32)
    h = jnp.maximum(h + b1_ref[...], 0.0)
    out_ref[...] = jnp.sum(h * w2_ref[...], axis=1, keepdims=True) + b2_ref[...]


def _mlp(u, i, w1uT, w1iT, b1, W2, b2):
    return pl.pallas_call(
        _mlp_body,
        grid=(_B // _BLK,),
        in_specs=[
            pl.BlockSpec((_BLK, _D), lambda b: (b, 0)),
            pl.BlockSpec((_BLK, _D), lambda b: (b, 0)),
            pl.BlockSpec((_D, _H), lambda b: (0, 0)),
            pl.BlockSpec((_D, _H), lambda b: (0, 0)),
            pl.BlockSpec((1, _H), lambda b: (0, 0)),
            pl.BlockSpec((1, _H), lambda b: (0, 0)),
            pl.BlockSpec((1, 1), lambda b: (0, 0)),
        ],
        out_specs=pl.BlockSpec((_BLK, 1), lambda b: (b, 0)),
        out_shape=jax.ShapeDtypeStruct((_B, 1), jnp.float32),
    )(u, i, w1uT, w1iT, b1, W2, b2)


def kernel(user_ids, item_ids, user_table, item_table, W1, b1, W2, b2):
    user_ids = user_ids.astype(jnp.int32)
    item_ids = item_ids.astype(jnp.int32)
    u_flat, i_flat = _sc_gather(user_ids, item_ids,
                                user_table.T, item_table.T)
    u = u_flat.reshape(_B, _D)
    i = i_flat.reshape(_B, _D)
    w1uT = W1[:, :_D].T
    w1iT = W1[:, _D:].T
    return _mlp(u, i, w1uT, w1iT, b1.reshape(1, _H), W2, b2.reshape(1, 1))
